# baseline (speedup 1.0000x reference)
import jax
import jax.numpy as jnp
from jax.experimental import pallas as pl
from jax.experimental.pallas import tpu as pltpu

F32 = jnp.float32
BF16 = jnp.bfloat16
B, PRIOR, FRAMES, POSE, PRED, CHUNK = 256, 60, 240, 768, 180, 10
EPS = 1e-5

BB_CONV = 8
BB_PATCH = 32
PKEEP = 16
PATCH_LO = 48
PATCH_H = 24

_VM = pltpu.CompilerParams(
    dimension_semantics=("arbitrary",),
    vmem_limit_bytes=56 * 1024 * 1024,
)


def _conv_post_body(x_ref, w1p_ref, w2p_ref,
                    cb1_ref, b2t_ref, s2_ref, t2_ref, pw1t_ref, pb1_ref,
                    pw2t_ref, pb2_ref, o_ref, p16_ref, pscr):
    zs4 = jnp.zeros((4, POSE), F32)
    zs12 = jnp.zeros((12, POSE), F32)
    for b in range(BB_CONV):
        xb = x_ref[b]
        z1 = jnp.zeros((PRIOR, 1), F32)
        xm = jnp.concatenate([z1, xb[:, :-1]], axis=1)
        xp = jnp.concatenate([xb[:, 1:], z1], axis=1)
        x3a = jnp.concatenate([xm, zs4, xb, zs4, xp, zs4],
                              axis=0).astype(BF16)
        c1 = jnp.dot(w1p_ref[...], x3a, preferred_element_type=F32)
        r1 = jnp.maximum(c1 + cb1_ref[...], 0.0)
        z2 = jnp.zeros((PRED, 1), F32)
        rm = jnp.concatenate([z2, r1[:, :-1]], axis=1)
        rp = jnp.concatenate([r1[:, 1:], z2], axis=1)
        r3a = jnp.concatenate([rm, zs12, r1, zs12, rp, zs12],
                              axis=0).astype(BF16)
        c2 = jnp.dot(w2p_ref[...], r3a, preferred_element_type=F32)
        pb = jnp.maximum(c2 + b2t_ref[...], 0.0) * s2_ref[...] + t2_ref[...]
        p16_ref[b] = pb[:PKEEP, :]
        pscr[b] = pb.astype(BF16)
    xflat = x_ref[...].reshape(BB_CONV * PRIOR, POSE).astype(BF16)
    hx = jnp.dot(xflat, pw1t_ref[...], preferred_element_type=F32) + pb1_ref[...]
    ox = jnp.dot(hx.astype(BF16), pw2t_ref[...], preferred_element_type=F32) + pb2_ref[...]
    o_ref[:, :PRIOR, :] = ox.reshape(BB_CONV, PRIOR, POSE)
    pflat = pscr[...].reshape(BB_CONV * PRED, POSE)
    hp2 = jnp.dot(pflat, pw1t_ref[...], preferred_element_type=F32) + pb1_ref[...]
    op = jnp.dot(hp2.astype(BF16), pw2t_ref[...], preferred_element_type=F32) + pb2_ref[...]
    o_ref[:, PRIOR:, :] = op.reshape(BB_CONV, PRED, POSE)


def _mem_body(xt_ref, spw1_ref, tmw1_ref, spb1_ref, spw2t_ref, spb2_ref,
              tmb1_ref, tmw2t_ref, tmb2_ref, mem_ref, mem2_ref, acc1, acc2):
    c = pl.program_id(0)

    @pl.when(c == 0)
    def _():
        acc1[...] = jnp.zeros_like(acc1)
        acc2[...] = jnp.zeros_like(acc2)

    xc = xt_ref[0].astype(BF16)
    acc1[...] += jax.lax.dot_general(
        xc, spw1_ref[...].astype(BF16), (((1,), (1,)), ((), ())),
        preferred_element_type=F32)
    acc2[...] += jax.lax.dot_general(
        xc, tmw1_ref[...].astype(BF16), (((1,), (1,)), ((), ())),
        preferred_element_type=F32)

    @pl.when(c == CHUNK - 1)
    def _():
        m1 = acc1[...] + spb1_ref[...]
        mem_ref[...] = (jnp.dot(m1, spw2t_ref[...], preferred_element_type=F32)
                        + spb2_ref[...])
        m2 = acc2[...] + tmb1_ref[...]
        mem2_ref[...] = (jnp.dot(m2, tmw2t_ref[...], preferred_element_type=F32)
                         + tmb2_ref[...])


def _gate_body(pc_ref, mem_ref, mem2_ref, tmmw1_ref, tmmb1_ref,
               tmmw2t_ref, tmmb2_ref, out_ref):
    mem = mem_ref[...]
    mem2 = mem2_ref[...]
    csps = []
    for c in range(CHUNK):
        pcc = pc_ref[:, c, :]
        sc = jnp.sum(mem * pcc, axis=1, keepdims=True)
        sig = jax.nn.sigmoid(sc)
        csps.append(sig * pcc + (1.0 - sig) * mem)
    acc = jnp.zeros((B, CHUNK), F32)
    for c in range(CHUNK):
        acc = acc + jnp.dot(csps[c], tmmw1_ref[c],
                            preferred_element_type=F32)
    penc = (jnp.dot(acc + tmmb1_ref[...], tmmw2t_ref[...],
                    preferred_element_type=F32) + tmmb2_ref[...])
    mmat = jax.lax.dot_general(mem2, penc, (((0,), (0,)), ((), ())),
                               preferred_element_type=F32)
    score2 = jnp.dot(mem2, mmat, preferred_element_type=F32)
    score2 = score2 - jnp.max(score2, axis=1, keepdims=True)
    es = jnp.exp(score2)
    soft = es / jnp.sum(es, axis=1, keepdims=True)
    for c in range(CHUNK):
        out_ref[:, c, :] = csps[c] * (1.0 + soft[:, c:c + 1])


def _patch_body(outa_ref, xt_ref, g_ref, p16_ref, pw1t_ref, pb1_ref,
                pw2t_ref, pb2_ref, o_ref):
    del outa_ref
    rows = jnp.concatenate(
        [xt_ref[:, PKEEP - (PRIOR - PATCH_LO):, :],
         g_ref[...],
         p16_ref[:, CHUNK:CHUNK + 2, :]],
        axis=1)
    flat = rows.reshape(BB_PATCH * PATCH_H, POSE).astype(BF16)
    h = jnp.dot(flat, pw1t_ref[...], preferred_element_type=F32) + pb1_ref[...]
    res = jnp.dot(h.astype(BF16), pw2t_ref[...], preferred_element_type=F32) + pb2_ref[...]
    o_ref[...] = res.reshape(BB_PATCH, PATCH_H, POSE)


def kernel(x, conv1_w, conv1_b, bn1_g, bn1_b, bn1_m, bn1_v,
           conv2_w, conv2_b, bn2_g, bn2_b, bn2_m, bn2_v,
           sp_w1, sp_b1, sp_w2, sp_b2,
           tmc_w1, tmc_b1, tmc_w2, tmc_b2,
           tmm_w1, tmm_b1, tmm_w2, tmm_b2,
           post_w1, post_b1, post_w2, post_b2):
    s1 = bn1_g * jax.lax.rsqrt(bn1_v + EPS)
    t1 = bn1_b - bn1_m * s1
    s2 = bn2_g * jax.lax.rsqrt(bn2_v + EPS)
    t2 = bn2_b - bn2_m * s2
    zp1 = jnp.zeros((PRED, 4), F32)
    w1p = jnp.concatenate(
        [conv1_w[:, :, 0], zp1, conv1_w[:, :, 1], zp1, conv1_w[:, :, 2], zp1],
        axis=1).astype(BF16)
    w2s = conv2_w * s1[None, :, None]
    zp2 = jnp.zeros((PRED, 12), F32)
    w2p = jnp.concatenate(
        [w2s[:, :, 0], zp2, w2s[:, :, 1], zp2, w2s[:, :, 2], zp2],
        axis=1).astype(BF16)
    tk = jnp.einsum('oik,i->ok', conv2_w, t1)
    cols = jnp.arange(POSE)[None, :]
    b2t = (conv2_b[:, None] + tk.sum(1)[:, None]
           - jnp.where(cols == 0, tk[:, 0:1], 0.0)
           - jnp.where(cols == POSE - 1, tk[:, 2:3], 0.0))
    bc = lambda v: jnp.broadcast_to(v[:, None], (PRED, POSE))
    cb1, s2b, t2b = bc(conv1_b), bc(s2), bc(t2)
    pw1t, pw2t = post_w1.T.astype(BF16), post_w2.T.astype(BF16)
    pb1, pb2 = post_b1.reshape(1, POSE), post_b2.reshape(1, POSE)

    full2 = lambda shape: pl.BlockSpec(shape, lambda i: (0, 0))
    outa, p16 = pl.pallas_call(
        _conv_post_body,
        grid=(B // BB_CONV,),
        in_specs=[
            pl.BlockSpec((BB_CONV, PRIOR, POSE), lambda i: (i, 0, 0)),
            full2((PRED, 192)), full2((PRED, 576)),
            full2((PRED, POSE)), full2((PRED, POSE)),
            full2((PRED, POSE)), full2((PRED, POSE)),
            full2((POSE, POSE)), full2((1, POSE)),
            full2((POSE, POSE)), full2((1, POSE)),
        ],
        out_specs=[
            pl.BlockSpec((BB_CONV, FRAMES, POSE), lambda i: (i, 0, 0)),
            pl.BlockSpec((BB_CONV, PKEEP, POSE), lambda i: (i, 0, 0)),
        ],
        out_shape=[
            jax.ShapeDtypeStruct((B, FRAMES, POSE), F32),
            jax.ShapeDtypeStruct((B, PKEEP, POSE), F32),
        ],
        scratch_shapes=[pltpu.VMEM((BB_CONV, PRED, POSE), BF16)],
        compiler_params=_VM,
        name="conv_post",
    )(x, w1p, w2p, cb1, b2t, s2b, t2b, pw1t, pb1, pw2t, pb2)

    xt = x[:, PRIOR - CHUNK:, :].transpose(1, 0, 2)
    mem, mem2 = pl.pallas_call(
        _mem_body,
        grid=(CHUNK,),
        in_specs=[
            pl.BlockSpec((1, B, POSE), lambda c: (c, 0, 0)),
            pl.BlockSpec((POSE, POSE), lambda c: (0, c)),
            pl.BlockSpec((POSE, POSE), lambda c: (0, c)),
            pl.BlockSpec((1, POSE), lambda c: (0, 0)),
            pl.BlockSpec((POSE, POSE), lambda c: (0, 0)),
            pl.BlockSpec((1, POSE), lambda c: (0, 0)),
            pl.BlockSpec((1, POSE), lambda c: (0, 0)),
            pl.BlockSpec((POSE, POSE), lambda c: (0, 0)),
            pl.BlockSpec((1, POSE), lambda c: (0, 0)),
        ],
        out_specs=[
            pl.BlockSpec((B, POSE), lambda c: (0, 0)),
            pl.BlockSpec((B, POSE), lambda c: (0, 0)),
        ],
        out_shape=[
            jax.ShapeDtypeStruct((B, POSE), F32),
            jax.ShapeDtypeStruct((B, POSE), F32),
        ],
        scratch_shapes=[
            pltpu.VMEM((B, POSE), F32),
            pltpu.VMEM((B, POSE), F32),
        ],
        compiler_params=_VM,
        name="tail_mem",
    )(xt, sp_w1, tmc_w1, sp_b1.reshape(1, POSE), sp_w2.T,
      sp_b2.reshape(1, POSE), tmc_b1.reshape(1, POSE), tmc_w2.T,
      tmc_b2.reshape(1, POSE))

    tmmw1r = tmm_w1.reshape(CHUNK, CHUNK, POSE).transpose(1, 2, 0)
    gated = pl.pallas_call(
        _gate_body,
        grid=(1,),
        in_specs=[
            pl.BlockSpec((B, PKEEP, POSE), lambda i: (0, 0, 0)),
            pl.BlockSpec((B, POSE), lambda i: (0, 0)),
            pl.BlockSpec((B, POSE), lambda i: (0, 0)),
            pl.BlockSpec((CHUNK, POSE, CHUNK), lambda i: (0, 0, 0)),
            pl.BlockSpec((1, CHUNK), lambda i: (0, 0)),
            pl.BlockSpec((CHUNK, CHUNK), lambda i: (0, 0)),
            pl.BlockSpec((1, CHUNK), lambda i: (0, 0)),
        ],
        out_specs=pl.BlockSpec((B, CHUNK, POSE), lambda i: (0, 0, 0)),
        out_shape=jax.ShapeDtypeStruct((B, CHUNK, POSE), F32),
        compiler_params=_VM,
        name="gating",
    )(p16, mem, mem2, tmmw1r, tmm_b1.reshape(1, CHUNK), tmm_w2.T,
      tmm_b2.reshape(1, CHUNK))

    xt16 = x[:, PRIOR - PKEEP:, :]
    out = pl.pallas_call(
        _patch_body,
        grid=(B // BB_PATCH,),
        in_specs=[
            pl.BlockSpec(memory_space=pl.ANY),
            pl.BlockSpec((BB_PATCH, PKEEP, POSE), lambda i: (i, 0, 0)),
            pl.BlockSpec((BB_PATCH, CHUNK, POSE), lambda i: (i, 0, 0)),
            pl.BlockSpec((BB_PATCH, PKEEP, POSE), lambda i: (i, 0, 0)),
            full2((POSE, POSE)), full2((1, POSE)),
            full2((POSE, POSE)), full2((1, POSE)),
        ],
        out_specs=pl.BlockSpec((BB_PATCH, PATCH_H, POSE),
                               lambda i: (i, PATCH_LO // PATCH_H, 0)),
        out_shape=jax.ShapeDtypeStruct((B, FRAMES, POSE), F32),
        input_output_aliases={0: 0},
        compiler_params=_VM,
        name="patch_chunk",
    )(outa, xt16, gated, p16, pw1t, pb1, pw2t, pb2)
    return out


# speedup vs baseline: 1.0344x; 1.0344x over previous
import jax
import jax.numpy as jnp
from jax.experimental import pallas as pl
from jax.experimental.pallas import tpu as pltpu

F32 = jnp.float32
BF16 = jnp.bfloat16
B, PRIOR, FRAMES, POSE, PRED, CHUNK = 256, 60, 240, 768, 180, 10
EPS = 1e-5

BB_CONV = 16
BB_PATCH = 32
PKEEP = 16
PATCH_LO = 48
PATCH_H = 24

_VM = pltpu.CompilerParams(
    dimension_semantics=("arbitrary",),
    vmem_limit_bytes=56 * 1024 * 1024,
)


def _conv_post_body(x_ref, w1a_ref, w1b_ref, w1c_ref, w2a_ref, w2b_ref, w2c_ref,
                    cb1_ref, b2t_ref, s2_ref, t2_ref, pw1t_ref, pb1_ref,
                    pw2t_ref, pb2_ref, o_ref, p16_ref, pscr):
    for b in range(BB_CONV):
        xb = x_ref[b]
        z1 = jnp.zeros((PRIOR, 1), F32)
        xm = jnp.concatenate([z1, xb[:, :-1]], axis=1).astype(BF16)
        xp = jnp.concatenate([xb[:, 1:], z1], axis=1).astype(BF16)
        c1 = (jnp.dot(w1a_ref[...], xm, preferred_element_type=F32)
              + jnp.dot(w1b_ref[...], xb.astype(BF16), preferred_element_type=F32)
              + jnp.dot(w1c_ref[...], xp, preferred_element_type=F32))
        r1 = jnp.maximum(c1 + cb1_ref[...], 0.0)
        z2 = jnp.zeros((PRED, 1), F32)
        rm = jnp.concatenate([z2, r1[:, :-1]], axis=1).astype(BF16)
        rp = jnp.concatenate([r1[:, 1:], z2], axis=1).astype(BF16)
        c2 = (jnp.dot(w2a_ref[...], rm, preferred_element_type=F32)
              + jnp.dot(w2b_ref[...], r1.astype(BF16), preferred_element_type=F32)
              + jnp.dot(w2c_ref[...], rp, preferred_element_type=F32))
        pb = jnp.maximum(c2 + b2t_ref[...], 0.0) * s2_ref[...] + t2_ref[...]
        p16_ref[b] = pb[:PKEEP, :]
        pscr[b] = pb.astype(BF16)
    xflat = x_ref[...].reshape(BB_CONV * PRIOR, POSE).astype(BF16)
    hx = jnp.dot(xflat, pw1t_ref[...], preferred_element_type=F32) + pb1_ref[...]
    ox = jnp.dot(hx.astype(BF16), pw2t_ref[...], preferred_element_type=F32) + pb2_ref[...]
    o_ref[:, :PRIOR, :] = ox.reshape(BB_CONV, PRIOR, POSE)
    pflat = pscr[...].reshape(BB_CONV * PRED, POSE)
    hp2 = jnp.dot(pflat, pw1t_ref[...], preferred_element_type=F32) + pb1_ref[...]
    op = jnp.dot(hp2.astype(BF16), pw2t_ref[...], preferred_element_type=F32) + pb2_ref[...]
    o_ref[:, PRIOR:, :] = op.reshape(BB_CONV, PRED, POSE)


def _mem_body(xt_ref, spw1_ref, tmw1_ref, spb1_ref, spw2t_ref, spb2_ref,
              tmb1_ref, tmw2t_ref, tmb2_ref, mem_ref, mem2_ref, acc1, acc2):
    c = pl.program_id(0)

    @pl.when(c == 0)
    def _():
        acc1[...] = jnp.zeros_like(acc1)
        acc2[...] = jnp.zeros_like(acc2)

    xc = xt_ref[0].astype(BF16)
    acc1[...] += jax.lax.dot_general(
        xc, spw1_ref[...].astype(BF16), (((1,), (1,)), ((), ())),
        preferred_element_type=F32)
    acc2[...] += jax.lax.dot_general(
        xc, tmw1_ref[...].astype(BF16), (((1,), (1,)), ((), ())),
        preferred_element_type=F32)

    @pl.when(c == CHUNK - 1)
    def _():
        m1 = acc1[...] + spb1_ref[...]
        mem_ref[...] = (jnp.dot(m1, spw2t_ref[...], preferred_element_type=F32)
                        + spb2_ref[...])
        m2 = acc2[...] + tmb1_ref[...]
        mem2_ref[...] = (jnp.dot(m2, tmw2t_ref[...], preferred_element_type=F32)
                         + tmb2_ref[...])


def _gate_body(pc_ref, mem_ref, mem2_ref, tmmw1_ref, tmmb1_ref,
               tmmw2t_ref, tmmb2_ref, out_ref):
    mem = mem_ref[...]
    mem2 = mem2_ref[...]
    csps = []
    for c in range(CHUNK):
        pcc = pc_ref[:, c, :]
        sc = jnp.sum(mem * pcc, axis=1, keepdims=True)
        sig = jax.nn.sigmoid(sc)
        csps.append(sig * pcc + (1.0 - sig) * mem)
    acc = jnp.zeros((B, CHUNK), F32)
    for c in range(CHUNK):
        acc = acc + jnp.dot(csps[c], tmmw1_ref[c],
                            preferred_element_type=F32)
    penc = (jnp.dot(acc + tmmb1_ref[...], tmmw2t_ref[...],
                    preferred_element_type=F32) + tmmb2_ref[...])
    mmat = jax.lax.dot_general(mem2, penc, (((0,), (0,)), ((), ())),
                               preferred_element_type=F32)
    score2 = jnp.dot(mem2, mmat, preferred_element_type=F32)
    score2 = score2 - jnp.max(score2, axis=1, keepdims=True)
    es = jnp.exp(score2)
    soft = es / jnp.sum(es, axis=1, keepdims=True)
    for c in range(CHUNK):
        out_ref[:, c, :] = csps[c] * (1.0 + soft[:, c:c + 1])


def _patch_body(outa_ref, xt_ref, g_ref, p16_ref, pw1t_ref, pb1_ref,
                pw2t_ref, pb2_ref, o_ref):
    del outa_ref
    rows = jnp.concatenate(
        [xt_ref[:, PKEEP - (PRIOR - PATCH_LO):, :],
         g_ref[...],
         p16_ref[:, CHUNK:CHUNK + 2, :]],
        axis=1)
    flat = rows.reshape(BB_PATCH * PATCH_H, POSE).astype(BF16)
    h = jnp.dot(flat, pw1t_ref[...], preferred_element_type=F32) + pb1_ref[...]
    res = jnp.dot(h.astype(BF16), pw2t_ref[...], preferred_element_type=F32) + pb2_ref[...]
    o_ref[...] = res.reshape(BB_PATCH, PATCH_H, POSE)


def kernel(x, conv1_w, conv1_b, bn1_g, bn1_b, bn1_m, bn1_v,
           conv2_w, conv2_b, bn2_g, bn2_b, bn2_m, bn2_v,
           sp_w1, sp_b1, sp_w2, sp_b2,
           tmc_w1, tmc_b1, tmc_w2, tmc_b2,
           tmm_w1, tmm_b1, tmm_w2, tmm_b2,
           post_w1, post_b1, post_w2, post_b2):
    s1 = bn1_g * jax.lax.rsqrt(bn1_v + EPS)
    t1 = bn1_b - bn1_m * s1
    s2 = bn2_g * jax.lax.rsqrt(bn2_v + EPS)
    t2 = bn2_b - bn2_m * s2
    w1a = conv1_w[:, :, 0].astype(BF16)
    w1b = conv1_w[:, :, 1].astype(BF16)
    w1c = conv1_w[:, :, 2].astype(BF16)
    w2s = conv2_w * s1[None, :, None]
    w2a = w2s[:, :, 0].astype(BF16)
    w2b = w2s[:, :, 1].astype(BF16)
    w2c = w2s[:, :, 2].astype(BF16)
    tk = jnp.einsum('oik,i->ok', conv2_w, t1)
    cols = jnp.arange(POSE)[None, :]
    b2t = (conv2_b[:, None] + tk.sum(1)[:, None]
           - jnp.where(cols == 0, tk[:, 0:1], 0.0)
           - jnp.where(cols == POSE - 1, tk[:, 2:3], 0.0))
    bc = lambda v: jnp.broadcast_to(v[:, None], (PRED, POSE))
    cb1, s2b, t2b = bc(conv1_b), bc(s2), bc(t2)
    pw1t, pw2t = post_w1.T.astype(BF16), post_w2.T.astype(BF16)
    pb1, pb2 = post_b1.reshape(1, POSE), post_b2.reshape(1, POSE)

    full2 = lambda shape: pl.BlockSpec(shape, lambda i: (0, 0))
    outa, p16 = pl.pallas_call(
        _conv_post_body,
        grid=(B // BB_CONV,),
        in_specs=[
            pl.BlockSpec((BB_CONV, PRIOR, POSE), lambda i: (i, 0, 0)),
            full2((PRED, PRIOR)), full2((PRED, PRIOR)), full2((PRED, PRIOR)),
            full2((PRED, PRED)), full2((PRED, PRED)), full2((PRED, PRED)),
            full2((PRED, POSE)), full2((PRED, POSE)),
            full2((PRED, POSE)), full2((PRED, POSE)),
            full2((POSE, POSE)), full2((1, POSE)),
            full2((POSE, POSE)), full2((1, POSE)),
        ],
        out_specs=[
            pl.BlockSpec((BB_CONV, FRAMES, POSE), lambda i: (i, 0, 0)),
            pl.BlockSpec((BB_CONV, PKEEP, POSE), lambda i: (i, 0, 0)),
        ],
        out_shape=[
            jax.ShapeDtypeStruct((B, FRAMES, POSE), F32),
            jax.ShapeDtypeStruct((B, PKEEP, POSE), F32),
        ],
        scratch_shapes=[pltpu.VMEM((BB_CONV, PRED, POSE), BF16)],
        compiler_params=_VM,
        name="conv_post",
    )(x, w1a, w1b, w1c, w2a, w2b, w2c, cb1, b2t, s2b, t2b, pw1t, pb1, pw2t, pb2)

    xt = x[:, PRIOR - CHUNK:, :].transpose(1, 0, 2)
    mem, mem2 = pl.pallas_call(
        _mem_body,
        grid=(CHUNK,),
        in_specs=[
            pl.BlockSpec((1, B, POSE), lambda c: (c, 0, 0)),
            pl.BlockSpec((POSE, POSE), lambda c: (0, c)),
            pl.BlockSpec((POSE, POSE), lambda c: (0, c)),
            pl.BlockSpec((1, POSE), lambda c: (0, 0)),
            pl.BlockSpec((POSE, POSE), lambda c: (0, 0)),
            pl.BlockSpec((1, POSE), lambda c: (0, 0)),
            pl.BlockSpec((1, POSE), lambda c: (0, 0)),
            pl.BlockSpec((POSE, POSE), lambda c: (0, 0)),
            pl.BlockSpec((1, POSE), lambda c: (0, 0)),
        ],
        out_specs=[
            pl.BlockSpec((B, POSE), lambda c: (0, 0)),
            pl.BlockSpec((B, POSE), lambda c: (0, 0)),
        ],
        out_shape=[
            jax.ShapeDtypeStruct((B, POSE), F32),
            jax.ShapeDtypeStruct((B, POSE), F32),
        ],
        scratch_shapes=[
            pltpu.VMEM((B, POSE), F32),
            pltpu.VMEM((B, POSE), F32),
        ],
        compiler_params=_VM,
        name="tail_mem",
    )(xt, sp_w1, tmc_w1, sp_b1.reshape(1, POSE), sp_w2.T,
      sp_b2.reshape(1, POSE), tmc_b1.reshape(1, POSE), tmc_w2.T,
      tmc_b2.reshape(1, POSE))

    tmmw1r = tmm_w1.reshape(CHUNK, CHUNK, POSE).transpose(1, 2, 0)
    gated = pl.pallas_call(
        _gate_body,
        grid=(1,),
        in_specs=[
            pl.BlockSpec((B, PKEEP, POSE), lambda i: (0, 0, 0)),
            pl.BlockSpec((B, POSE), lambda i: (0, 0)),
            pl.BlockSpec((B, POSE), lambda i: (0, 0)),
            pl.BlockSpec((CHUNK, POSE, CHUNK), lambda i: (0, 0, 0)),
            pl.BlockSpec((1, CHUNK), lambda i: (0, 0)),
            pl.BlockSpec((CHUNK, CHUNK), lambda i: (0, 0)),
            pl.BlockSpec((1, CHUNK), lambda i: (0, 0)),
        ],
        out_specs=pl.BlockSpec((B, CHUNK, POSE), lambda i: (0, 0, 0)),
        out_shape=jax.ShapeDtypeStruct((B, CHUNK, POSE), F32),
        compiler_params=_VM,
        name="gating",
    )(p16, mem, mem2, tmmw1r, tmm_b1.reshape(1, CHUNK), tmm_w2.T,
      tmm_b2.reshape(1, CHUNK))

    xt16 = x[:, PRIOR - PKEEP:, :]
    out = pl.pallas_call(
        _patch_body,
        grid=(B // BB_PATCH,),
        in_specs=[
            pl.BlockSpec(memory_space=pl.ANY),
            pl.BlockSpec((BB_PATCH, PKEEP, POSE), lambda i: (i, 0, 0)),
            pl.BlockSpec((BB_PATCH, CHUNK, POSE), lambda i: (i, 0, 0)),
            pl.BlockSpec((BB_PATCH, PKEEP, POSE), lambda i: (i, 0, 0)),
            full2((POSE, POSE)), full2((1, POSE)),
            full2((POSE, POSE)), full2((1, POSE)),
        ],
        out_specs=pl.BlockSpec((BB_PATCH, PATCH_H, POSE),
                               lambda i: (i, PATCH_LO // PATCH_H, 0)),
        out_shape=jax.ShapeDtypeStruct((B, FRAMES, POSE), F32),
        input_output_aliases={0: 0},
        compiler_params=_VM,
        name="patch_chunk",
    )(outa, xt16, gated, p16, pw1t, pb1, pw2t, pb2)
    return out


# speedup vs baseline: 1.0367x; 1.0022x over previous
import jax
import jax.numpy as jnp
from jax.experimental import pallas as pl
from jax.experimental.pallas import tpu as pltpu

F32 = jnp.float32
BF16 = jnp.bfloat16
B, PRIOR, FRAMES, POSE, PRED, CHUNK = 256, 60, 240, 768, 180, 10
EPS = 1e-5

BB_CONV = 16
BB_PATCH = 32
PKEEP = 16
PATCH_LO = 48
PATCH_H = 24

_VM = pltpu.CompilerParams(
    dimension_semantics=("arbitrary",),
    vmem_limit_bytes=56 * 1024 * 1024,
)


def _conv_post_body(x_ref, w1a_ref, w1b_ref, w1c_ref, w2a_ref, w2b_ref, w2c_ref,
                    cb1_ref, b2t_ref, s2_ref, t2_ref, pw1t_ref, pb1_ref,
                    pw2t_ref, pb2_ref, o_ref, p16_ref, pscr):
    for b in range(BB_CONV):
        xb = x_ref[b]
        z1 = jnp.zeros((PRIOR, 1), F32)
        xm = jnp.concatenate([z1, xb[:, :-1]], axis=1).astype(BF16)
        xp = jnp.concatenate([xb[:, 1:], z1], axis=1).astype(BF16)
        c1 = (jnp.dot(w1a_ref[...], xm, preferred_element_type=F32)
              + jnp.dot(w1b_ref[...], xb.astype(BF16), preferred_element_type=F32)
              + jnp.dot(w1c_ref[...], xp, preferred_element_type=F32))
        r1 = jnp.maximum(c1 + cb1_ref[...], 0.0)
        z2 = jnp.zeros((PRED, 1), F32)
        rm = jnp.concatenate([z2, r1[:, :-1]], axis=1).astype(BF16)
        rp = jnp.concatenate([r1[:, 1:], z2], axis=1).astype(BF16)
        c2 = (jnp.dot(w2a_ref[...], rm, preferred_element_type=F32)
              + jnp.dot(w2b_ref[...], r1.astype(BF16), preferred_element_type=F32)
              + jnp.dot(w2c_ref[...], rp, preferred_element_type=F32))
        pb = jnp.maximum(c2 + b2t_ref[...], 0.0) * s2_ref[...] + t2_ref[...]
        p16_ref[b] = pb[:PKEEP, :]
        pscr[b] = pb.astype(BF16)
    xflat = x_ref[...].reshape(BB_CONV * PRIOR, POSE).astype(BF16)
    hx = jnp.dot(xflat, pw1t_ref[...], preferred_element_type=F32) + pb1_ref[...]
    ox = jnp.dot(hx.astype(BF16), pw2t_ref[...], preferred_element_type=F32) + pb2_ref[...]
    o_ref[:, :PRIOR, :] = ox.reshape(BB_CONV, PRIOR, POSE)
    for g in range(4):
        gl, gh = g * (BB_CONV // 4), (g + 1) * (BB_CONV // 4)
        pf = pscr[gl:gh].reshape((BB_CONV // 4) * PRED, POSE)
        hp2 = jnp.dot(pf, pw1t_ref[...], preferred_element_type=F32) + pb1_ref[...]
        op = jnp.dot(hp2.astype(BF16), pw2t_ref[...], preferred_element_type=F32) + pb2_ref[...]
        o_ref[gl:gh, PRIOR:, :] = op.reshape(BB_CONV // 4, PRED, POSE)


def _mem_body(xt_ref, spw1_ref, tmw1_ref, spb1_ref, spw2t_ref, spb2_ref,
              tmb1_ref, tmw2t_ref, tmb2_ref, mem_ref, mem2_ref, acc1, acc2):
    c = pl.program_id(0)

    @pl.when(c == 0)
    def _():
        acc1[...] = jnp.zeros_like(acc1)
        acc2[...] = jnp.zeros_like(acc2)

    xc = xt_ref[0].astype(BF16)
    acc1[...] += jax.lax.dot_general(
        xc, spw1_ref[...].astype(BF16), (((1,), (1,)), ((), ())),
        preferred_element_type=F32)
    acc2[...] += jax.lax.dot_general(
        xc, tmw1_ref[...].astype(BF16), (((1,), (1,)), ((), ())),
        preferred_element_type=F32)

    @pl.when(c == CHUNK - 1)
    def _():
        m1 = acc1[...] + spb1_ref[...]
        mem_ref[...] = (jnp.dot(m1, spw2t_ref[...], preferred_element_type=F32)
                        + spb2_ref[...])
        m2 = acc2[...] + tmb1_ref[...]
        mem2_ref[...] = (jnp.dot(m2, tmw2t_ref[...], preferred_element_type=F32)
                         + tmb2_ref[...])


def _gate_body(pc_ref, mem_ref, mem2_ref, tmmw1_ref, tmmb1_ref,
               tmmw2t_ref, tmmb2_ref, out_ref):
    mem = mem_ref[...]
    mem2 = mem2_ref[...]
    csps = []
    for c in range(CHUNK):
        pcc = pc_ref[:, c, :]
        sc = jnp.sum(mem * pcc, axis=1, keepdims=True)
        sig = jax.nn.sigmoid(sc)
        csps.append(sig * pcc + (1.0 - sig) * mem)
    acc = jnp.zeros((B, CHUNK), F32)
    for c in range(CHUNK):
        acc = acc + jnp.dot(csps[c], tmmw1_ref[c],
                            preferred_element_type=F32)
    penc = (jnp.dot(acc + tmmb1_ref[...], tmmw2t_ref[...],
                    preferred_element_type=F32) + tmmb2_ref[...])
    mmat = jax.lax.dot_general(mem2, penc, (((0,), (0,)), ((), ())),
                               preferred_element_type=F32)
    score2 = jnp.dot(mem2, mmat, preferred_element_type=F32)
    score2 = score2 - jnp.max(score2, axis=1, keepdims=True)
    es = jnp.exp(score2)
    soft = es / jnp.sum(es, axis=1, keepdims=True)
    for c in range(CHUNK):
        out_ref[:, c, :] = csps[c] * (1.0 + soft[:, c:c + 1])


def _patch_body(outa_ref, xt_ref, g_ref, p16_ref, pw1t_ref, pb1_ref,
                pw2t_ref, pb2_ref, o_ref):
    del outa_ref
    rows = jnp.concatenate(
        [xt_ref[:, PKEEP - (PRIOR - PATCH_LO):, :],
         g_ref[...],
         p16_ref[:, CHUNK:CHUNK + 2, :]],
        axis=1)
    flat = rows.reshape(BB_PATCH * PATCH_H, POSE).astype(BF16)
    h = jnp.dot(flat, pw1t_ref[...], preferred_element_type=F32) + pb1_ref[...]
    res = jnp.dot(h.astype(BF16), pw2t_ref[...], preferred_element_type=F32) + pb2_ref[...]
    o_ref[...] = res.reshape(BB_PATCH, PATCH_H, POSE)


def kernel(x, conv1_w, conv1_b, bn1_g, bn1_b, bn1_m, bn1_v,
           conv2_w, conv2_b, bn2_g, bn2_b, bn2_m, bn2_v,
           sp_w1, sp_b1, sp_w2, sp_b2,
           tmc_w1, tmc_b1, tmc_w2, tmc_b2,
           tmm_w1, tmm_b1, tmm_w2, tmm_b2,
           post_w1, post_b1, post_w2, post_b2):
    s1 = bn1_g * jax.lax.rsqrt(bn1_v + EPS)
    t1 = bn1_b - bn1_m * s1
    s2 = bn2_g * jax.lax.rsqrt(bn2_v + EPS)
    t2 = bn2_b - bn2_m * s2
    w1a = conv1_w[:, :, 0].astype(BF16)
    w1b = conv1_w[:, :, 1].astype(BF16)
    w1c = conv1_w[:, :, 2].astype(BF16)
    w2s = conv2_w * s1[None, :, None]
    w2a = w2s[:, :, 0].astype(BF16)
    w2b = w2s[:, :, 1].astype(BF16)
    w2c = w2s[:, :, 2].astype(BF16)
    tk = jnp.einsum('oik,i->ok', conv2_w, t1)
    cols = jnp.arange(POSE)[None, :]
    b2t = (conv2_b[:, None] + tk.sum(1)[:, None]
           - jnp.where(cols == 0, tk[:, 0:1], 0.0)
           - jnp.where(cols == POSE - 1, tk[:, 2:3], 0.0))
    bc = lambda v: jnp.broadcast_to(v[:, None], (PRED, POSE))
    cb1, s2b, t2b = bc(conv1_b), bc(s2), bc(t2)
    pw1t, pw2t = post_w1.T.astype(BF16), post_w2.T.astype(BF16)
    pb1, pb2 = post_b1.reshape(1, POSE), post_b2.reshape(1, POSE)

    full2 = lambda shape: pl.BlockSpec(shape, lambda i: (0, 0))
    outa, p16 = pl.pallas_call(
        _conv_post_body,
        grid=(B // BB_CONV,),
        in_specs=[
            pl.BlockSpec((BB_CONV, PRIOR, POSE), lambda i: (i, 0, 0)),
            full2((PRED, PRIOR)), full2((PRED, PRIOR)), full2((PRED, PRIOR)),
            full2((PRED, PRED)), full2((PRED, PRED)), full2((PRED, PRED)),
            full2((PRED, POSE)), full2((PRED, POSE)),
            full2((PRED, POSE)), full2((PRED, POSE)),
            full2((POSE, POSE)), full2((1, POSE)),
            full2((POSE, POSE)), full2((1, POSE)),
        ],
        out_specs=[
            pl.BlockSpec((BB_CONV, FRAMES, POSE), lambda i: (i, 0, 0)),
            pl.BlockSpec((BB_CONV, PKEEP, POSE), lambda i: (i, 0, 0)),
        ],
        out_shape=[
            jax.ShapeDtypeStruct((B, FRAMES, POSE), F32),
            jax.ShapeDtypeStruct((B, PKEEP, POSE), F32),
        ],
        scratch_shapes=[pltpu.VMEM((BB_CONV, PRED, POSE), BF16)],
        compiler_params=_VM,
        name="conv_post",
    )(x, w1a, w1b, w1c, w2a, w2b, w2c, cb1, b2t, s2b, t2b, pw1t, pb1, pw2t, pb2)

    xt = x[:, PRIOR - CHUNK:, :].transpose(1, 0, 2)
    mem, mem2 = pl.pallas_call(
        _mem_body,
        grid=(CHUNK,),
        in_specs=[
            pl.BlockSpec((1, B, POSE), lambda c: (c, 0, 0)),
            pl.BlockSpec((POSE, POSE), lambda c: (0, c)),
            pl.BlockSpec((POSE, POSE), lambda c: (0, c)),
            pl.BlockSpec((1, POSE), lambda c: (0, 0)),
            pl.BlockSpec((POSE, POSE), lambda c: (0, 0)),
            pl.BlockSpec((1, POSE), lambda c: (0, 0)),
            pl.BlockSpec((1, POSE), lambda c: (0, 0)),
            pl.BlockSpec((POSE, POSE), lambda c: (0, 0)),
            pl.BlockSpec((1, POSE), lambda c: (0, 0)),
        ],
        out_specs=[
            pl.BlockSpec((B, POSE), lambda c: (0, 0)),
            pl.BlockSpec((B, POSE), lambda c: (0, 0)),
        ],
        out_shape=[
            jax.ShapeDtypeStruct((B, POSE), F32),
            jax.ShapeDtypeStruct((B, POSE), F32),
        ],
        scratch_shapes=[
            pltpu.VMEM((B, POSE), F32),
            pltpu.VMEM((B, POSE), F32),
        ],
        compiler_params=_VM,
        name="tail_mem",
    )(xt, sp_w1, tmc_w1, sp_b1.reshape(1, POSE), sp_w2.T,
      sp_b2.reshape(1, POSE), tmc_b1.reshape(1, POSE), tmc_w2.T,
      tmc_b2.reshape(1, POSE))

    tmmw1r = tmm_w1.reshape(CHUNK, CHUNK, POSE).transpose(1, 2, 0)
    gated = pl.pallas_call(
        _gate_body,
        grid=(1,),
        in_specs=[
            pl.BlockSpec((B, PKEEP, POSE), lambda i: (0, 0, 0)),
            pl.BlockSpec((B, POSE), lambda i: (0, 0)),
            pl.BlockSpec((B, POSE), lambda i: (0, 0)),
            pl.BlockSpec((CHUNK, POSE, CHUNK), lambda i: (0, 0, 0)),
            pl.BlockSpec((1, CHUNK), lambda i: (0, 0)),
            pl.BlockSpec((CHUNK, CHUNK), lambda i: (0, 0)),
            pl.BlockSpec((1, CHUNK), lambda i: (0, 0)),
        ],
        out_specs=pl.BlockSpec((B, CHUNK, POSE), lambda i: (0, 0, 0)),
        out_shape=jax.ShapeDtypeStruct((B, CHUNK, POSE), F32),
        compiler_params=_VM,
        name="gating",
    )(p16, mem, mem2, tmmw1r, tmm_b1.reshape(1, CHUNK), tmm_w2.T,
      tmm_b2.reshape(1, CHUNK))

    xt16 = x[:, PRIOR - PKEEP:, :]
    out = pl.pallas_call(
        _patch_body,
        grid=(B // BB_PATCH,),
        in_specs=[
            pl.BlockSpec(memory_space=pl.ANY),
            pl.BlockSpec((BB_PATCH, PKEEP, POSE), lambda i: (i, 0, 0)),
            pl.BlockSpec((BB_PATCH, CHUNK, POSE), lambda i: (i, 0, 0)),
            pl.BlockSpec((BB_PATCH, PKEEP, POSE), lambda i: (i, 0, 0)),
            full2((POSE, POSE)), full2((1, POSE)),
            full2((POSE, POSE)), full2((1, POSE)),
        ],
        out_specs=pl.BlockSpec((BB_PATCH, PATCH_H, POSE),
                               lambda i: (i, PATCH_LO // PATCH_H, 0)),
        out_shape=jax.ShapeDtypeStruct((B, FRAMES, POSE), F32),
        input_output_aliases={0: 0},
        compiler_params=_VM,
        name="patch_chunk",
    )(outa, xt16, gated, p16, pw1t, pb1, pw2t, pb2)
    return out
